# baseline (speedup 1.0000x reference)
import functools

import jax
import jax.numpy as jnp
from jax.experimental import pallas as pl
from jax.experimental.pallas import tpu as pltpu

B, S, H = 2, 2048, 1024
E, K, F = 8, 2, 256
SHARED_F = 512
T = B * S


def _moe_block(x_ref, gwt_ref, wg_ref, wu_ref, wd_ref, wsg_ref, wsu_ref,
               wsd_ref, out_ref):
    x32 = x_ref[...]
    bt = x32.shape[0]

    logits = jnp.dot(x32, gwt_ref[...], preferred_element_type=jnp.float32)
    m = jnp.max(logits, axis=-1, keepdims=True)
    p = jnp.exp(logits - m)
    scores = p / jnp.sum(p, axis=-1, keepdims=True)
    lane = jax.lax.broadcasted_iota(jnp.int32, (bt, E), 1)
    i1 = jnp.argmax(scores, axis=-1)[:, None]
    m1 = jnp.max(scores, axis=-1, keepdims=True)
    masked = jnp.where(lane == i1, -1.0, scores)
    i2 = jnp.argmax(masked, axis=-1)[:, None]
    m2 = jnp.max(masked, axis=-1, keepdims=True)
    c = jnp.where(lane == i1, m1, 0.0) + jnp.where(lane == i2, m2, 0.0)

    sg = jnp.dot(x32, wsg_ref[...], preferred_element_type=jnp.float32)
    su = jnp.dot(x32, wsu_ref[...], preferred_element_type=jnp.float32)
    inter_s = sg * jax.nn.sigmoid(sg) * su
    acc = jnp.dot(inter_s, wsd_ref[...], preferred_element_type=jnp.float32)

    for e in range(E):
        g = jnp.dot(x32, wg_ref[e], preferred_element_type=jnp.float32)
        u = jnp.dot(x32, wu_ref[e], preferred_element_type=jnp.float32)
        he_in = c[:, e:e + 1] * (g * jax.nn.sigmoid(g) * u)
        acc = acc + jnp.dot(he_in, wd_ref[e],
                            preferred_element_type=jnp.float32)

    out_ref[...] = acc


@functools.partial(jax.jit, static_argnames=("bt",))
def _moe(x, gwt, wg, wu, wd, wsg, wsu, wsd, bt=128):
    grid = (T // bt,)
    return pl.pallas_call(
        _moe_block,
        grid=grid,
        in_specs=[
            pl.BlockSpec((bt, H), lambda i: (i, 0)),
            pl.BlockSpec((H, E), lambda i: (0, 0)),
            pl.BlockSpec((E, H, F), lambda i: (0, 0, 0)),
            pl.BlockSpec((E, H, F), lambda i: (0, 0, 0)),
            pl.BlockSpec((E, F, H), lambda i: (0, 0, 0)),
            pl.BlockSpec((H, SHARED_F), lambda i: (0, 0)),
            pl.BlockSpec((H, SHARED_F), lambda i: (0, 0)),
            pl.BlockSpec((SHARED_F, H), lambda i: (0, 0)),
        ],
        out_specs=pl.BlockSpec((bt, H), lambda i: (i, 0)),
        out_shape=jax.ShapeDtypeStruct((T, H), jnp.float32),
    )(x, gwt, wg, wu, wd, wsg, wsu, wsd)


def kernel(hidden_states, gate_w, w_gate, w_up, w_down, ws_gate, ws_up,
           ws_down):
    x = hidden_states.reshape(-1, H)
    out = _moe(x, gate_w.T, w_gate, w_up, w_down, ws_gate, ws_up, ws_down)
    return out.reshape(B, S, H)


# speedup vs baseline: 1.4389x; 1.4389x over previous
import functools

import jax
import jax.numpy as jnp
from jax.experimental import pallas as pl
from jax.experimental.pallas import tpu as pltpu

B, S, H = 2, 2048, 1024
E, K, F = 8, 2, 256
SHARED_F = 512
T = B * S


def _moe_block(x_ref, gwt_ref, wg_ref, wu_ref, wd_ref, wsg_ref, wsu_ref,
               wsd_ref, out_ref):
    x32 = x_ref[...]
    bt = x32.shape[0]

    logits = jnp.dot(x32, gwt_ref[...], preferred_element_type=jnp.float32)
    m = jnp.max(logits, axis=-1, keepdims=True)
    p = jnp.exp(logits - m)
    scores = p / jnp.sum(p, axis=-1, keepdims=True)
    lane = jax.lax.broadcasted_iota(jnp.int32, (bt, E), 1)
    i1 = jnp.argmax(scores, axis=-1)[:, None]
    m1 = jnp.max(scores, axis=-1, keepdims=True)
    masked = jnp.where(lane == i1, -1.0, scores)
    i2 = jnp.argmax(masked, axis=-1)[:, None]
    m2 = jnp.max(masked, axis=-1, keepdims=True)
    c = jnp.where(lane == i1, m1, 0.0) + jnp.where(lane == i2, m2, 0.0)

    sg = jnp.dot(x32, wsg_ref[...], preferred_element_type=jnp.float32)
    su = jnp.dot(x32, wsu_ref[...], preferred_element_type=jnp.float32)
    inter_s = sg * jax.nn.sigmoid(sg) * su
    acc = jnp.dot(inter_s, wsd_ref[...], preferred_element_type=jnp.float32)

    for e in range(E):
        g = jnp.dot(x32, wg_ref[e], preferred_element_type=jnp.float32)
        u = jnp.dot(x32, wu_ref[e], preferred_element_type=jnp.float32)
        he_in = c[:, e:e + 1] * (g * jax.nn.sigmoid(g) * u)
        acc = acc + jnp.dot(he_in, wd_ref[e],
                            preferred_element_type=jnp.float32)

    out_ref[...] = acc


@functools.partial(jax.jit, static_argnames=("bt",))
def _moe(x, gwt, wg, wu, wd, wsg, wsu, wsd, bt=512):
    grid = (T // bt,)
    return pl.pallas_call(
        _moe_block,
        grid=grid,
        in_specs=[
            pl.BlockSpec((bt, H), lambda i: (i, 0)),
            pl.BlockSpec((H, E), lambda i: (0, 0)),
            pl.BlockSpec((E, H, F), lambda i: (0, 0, 0)),
            pl.BlockSpec((E, H, F), lambda i: (0, 0, 0)),
            pl.BlockSpec((E, F, H), lambda i: (0, 0, 0)),
            pl.BlockSpec((H, SHARED_F), lambda i: (0, 0)),
            pl.BlockSpec((H, SHARED_F), lambda i: (0, 0)),
            pl.BlockSpec((SHARED_F, H), lambda i: (0, 0)),
        ],
        out_specs=pl.BlockSpec((bt, H), lambda i: (i, 0)),
        out_shape=jax.ShapeDtypeStruct((T, H), jnp.float32),
    )(x, gwt, wg, wu, wd, wsg, wsu, wsd)


def kernel(hidden_states, gate_w, w_gate, w_up, w_down, ws_gate, ws_up,
           ws_down):
    x = hidden_states.reshape(-1, H)
    out = _moe(x, gate_w.T, w_gate, w_up, w_down, ws_gate, ws_up, ws_down)
    return out.reshape(B, S, H)


# speedup vs baseline: 1.4583x; 1.0135x over previous
import functools

import jax
import jax.numpy as jnp
from jax.experimental import pallas as pl
from jax.experimental.pallas import tpu as pltpu

B, S, H = 2, 2048, 1024
E, K, F = 8, 2, 256
SHARED_F = 512
T = B * S


def _moe_block(x_ref, gwt_ref, wg_ref, wu_ref, wd_ref, wsg_ref, wsu_ref,
               wsd_ref, out_ref):
    x32 = x_ref[...]
    bt = x32.shape[0]

    logits = jnp.dot(x32, gwt_ref[...], preferred_element_type=jnp.float32)
    m = jnp.max(logits, axis=-1, keepdims=True)
    p = jnp.exp(logits - m)
    scores = p / jnp.sum(p, axis=-1, keepdims=True)
    lane = jax.lax.broadcasted_iota(jnp.int32, (bt, E), 1)
    i1 = jnp.argmax(scores, axis=-1)[:, None]
    m1 = jnp.max(scores, axis=-1, keepdims=True)
    masked = jnp.where(lane == i1, -1.0, scores)
    i2 = jnp.argmax(masked, axis=-1)[:, None]
    m2 = jnp.max(masked, axis=-1, keepdims=True)
    c = jnp.where(lane == i1, m1, 0.0) + jnp.where(lane == i2, m2, 0.0)

    sg = jnp.dot(x32, wsg_ref[...], preferred_element_type=jnp.float32)
    su = jnp.dot(x32, wsu_ref[...], preferred_element_type=jnp.float32)
    inter_s = sg * jax.nn.sigmoid(sg) * su
    acc = jnp.dot(inter_s, wsd_ref[...], preferred_element_type=jnp.float32)

    for e in range(E):
        g = jnp.dot(x32, wg_ref[e], preferred_element_type=jnp.float32)
        u = jnp.dot(x32, wu_ref[e], preferred_element_type=jnp.float32)
        he_in = c[:, e:e + 1] * (g * jax.nn.sigmoid(g) * u)
        acc = acc + jnp.dot(he_in, wd_ref[e],
                            preferred_element_type=jnp.float32)

    out_ref[...] = acc


@functools.partial(jax.jit, static_argnames=("bt",))
def _moe(x, gwt, wg, wu, wd, wsg, wsu, wsd, bt=1024):
    grid = (T // bt,)
    return pl.pallas_call(
        _moe_block,
        grid=grid,
        in_specs=[
            pl.BlockSpec((bt, H), lambda i: (i, 0)),
            pl.BlockSpec((H, E), lambda i: (0, 0)),
            pl.BlockSpec((E, H, F), lambda i: (0, 0, 0)),
            pl.BlockSpec((E, H, F), lambda i: (0, 0, 0)),
            pl.BlockSpec((E, F, H), lambda i: (0, 0, 0)),
            pl.BlockSpec((H, SHARED_F), lambda i: (0, 0)),
            pl.BlockSpec((H, SHARED_F), lambda i: (0, 0)),
            pl.BlockSpec((SHARED_F, H), lambda i: (0, 0)),
        ],
        out_specs=pl.BlockSpec((bt, H), lambda i: (i, 0)),
        out_shape=jax.ShapeDtypeStruct((T, H), jnp.float32),
    )(x, gwt, wg, wu, wd, wsg, wsu, wsd)


def kernel(hidden_states, gate_w, w_gate, w_up, w_down, ws_gate, ws_up,
           ws_down):
    x = hidden_states.reshape(-1, H)
    out = _moe(x, gate_w.T, w_gate, w_up, w_down, ws_gate, ws_up, ws_down)
    return out.reshape(B, S, H)
